# baseline (speedup 1.0000x reference)
import jax
import jax.numpy as jnp
from jax import lax
from jax.experimental import pallas as pl
from jax.experimental.pallas import tpu as pltpu, tpu_sc as plsc

B = 16384
D = 128
DIM = 2 * D
N_SYN = 512
N_SEM = 1024
CTX_GATE_STRENGTH = 2.0
COMMITMENT_COST = 0.25

BLOCK_B = 2048


def _vq_block(zb, zsq, cwT, b, csq, k):
    p = jax.lax.dot_general(zb, cwT, (((1,), (0,)), ((), ())),
                            preferred_element_type=jnp.float32)
    zc = p[:, :k]
    d = (zsq + csq) - 2.0 * zc
    logits = p[:, k:] + b
    m = jnp.max(logits, axis=1, keepdims=True)
    e = jnp.exp(logits - m)
    bias = CTX_GATE_STRENGTH * (e / jnp.sum(e, axis=1, keepdims=True))
    dtot = d - bias
    dmin = jnp.min(dtot, axis=1, keepdims=True)
    lane = jax.lax.broadcasted_iota(jnp.int32, dtot.shape, 1)
    idx = jnp.min(jnp.where(dtot == dmin, lane, k), axis=1)
    bias_at = jnp.sum(jnp.where(lane == idx[:, None], bias, 0.0), axis=1)
    loss_part = jnp.sum(dmin[:, 0] + bias_at)
    return idx, loss_part


def _tc_kernel(zfr_ref, zfi_ref, zsr_ref, zsi_ref, zfsq_ref, zssq_ref,
               cwT_syn_ref, b_syn_ref, csq_syn_ref,
               cwT_sem_ref, b_sem_ref, csq_sem_ref,
               idx_syn_ref, idx_sem_ref, loss_ref):
    zf = jnp.concatenate([zfr_ref[...], zfi_ref[...]], axis=1)
    zs = jnp.concatenate([zsr_ref[...], zsi_ref[...]], axis=1)
    i_syn, l_syn = _vq_block(zf.astype(jnp.bfloat16), zfsq_ref[...],
                             cwT_syn_ref[...], b_syn_ref[...],
                             csq_syn_ref[...], N_SYN)
    i_sem, l_sem = _vq_block(zs.astype(jnp.bfloat16), zssq_ref[...],
                             cwT_sem_ref[...], b_sem_ref[...],
                             csq_sem_ref[...], N_SEM)
    idx_syn_ref[...] = i_syn[:, None]
    idx_sem_ref[...] = i_sem[:, None]

    @pl.when(pl.program_id(0) == 0)
    def _init():
        loss_ref[...] = jnp.zeros_like(loss_ref)

    loss_ref[...] += l_syn + l_sem


_SC_CHUNK = 64


def _sc_gather(tab_re_hbm, tab_im_hbm, idx_hbm, out_re_hbm, out_im_hbm,
               idx_v, rows_v, sem):
    info = plsc.get_sparse_core_info()
    nw = info.num_cores * info.num_subcores
    wid = lax.axis_index("s") * info.num_cores + lax.axis_index("c")
    per_w = B // nw
    base = wid * per_w
    nchunks = per_w // _SC_CHUNK
    for c in range(nchunks):
        off = base + c * _SC_CHUNK
        pltpu.sync_copy(idx_hbm.at[pl.ds(off, _SC_CHUNK)], idx_v)
        pltpu.async_copy(tab_re_hbm.at[idx_v], rows_v, sem).wait()
        pltpu.sync_copy(rows_v, out_re_hbm.at[pl.ds(off, _SC_CHUNK)])
        pltpu.async_copy(tab_im_hbm.at[idx_v], rows_v, sem).wait()
        pltpu.sync_copy(rows_v, out_im_hbm.at[pl.ds(off, _SC_CHUNK)])


def kernel(z_fast_real, z_fast_imag, z_slow_real, z_slow_imag,
           cb_syn, cb_sem, W_ctx_syn, b_ctx_syn, W_ctx_sem, b_ctx_sem):
    cwT_syn = jnp.concatenate([cb_syn.T, W_ctx_syn.T], axis=1).astype(jnp.bfloat16)
    cwT_sem = jnp.concatenate([cb_sem.T, W_ctx_sem.T], axis=1).astype(jnp.bfloat16)
    csq_syn = jnp.sum(cb_syn ** 2, axis=1)[None, :]
    csq_sem = jnp.sum(cb_sem ** 2, axis=1)[None, :]
    zfsq = jnp.sum(jnp.concatenate([z_fast_real, z_fast_imag], axis=1) ** 2,
                   axis=1, keepdims=True)
    zssq = jnp.sum(jnp.concatenate([z_slow_real, z_slow_imag], axis=1) ** 2,
                   axis=1, keepdims=True)
    b_syn = b_ctx_syn[None, :]
    b_sem = b_ctx_sem[None, :]

    nb = B // BLOCK_B
    half_spec = pl.BlockSpec((BLOCK_B, D), lambda i: (i, 0))
    sq_spec = pl.BlockSpec((BLOCK_B, 1), lambda i: (i, 0))
    full = lambda shape: pl.BlockSpec(shape, lambda i: (0,) * len(shape))

    idx_syn, idx_sem, loss_acc = pl.pallas_call(
        _tc_kernel,
        grid=(nb,),
        in_specs=[
            half_spec, half_spec, half_spec, half_spec, sq_spec, sq_spec,
            full((DIM, 2 * N_SYN)), full((1, N_SYN)), full((1, N_SYN)),
            full((DIM, 2 * N_SEM)), full((1, N_SEM)), full((1, N_SEM)),
        ],
        out_specs=(
            sq_spec,
            sq_spec,
            pl.BlockSpec((1, 1), lambda i: (0, 0)),
        ),
        out_shape=(
            jax.ShapeDtypeStruct((B, 1), jnp.int32),
            jax.ShapeDtypeStruct((B, 1), jnp.int32),
            jax.ShapeDtypeStruct((1, 1), jnp.float32),
        ),
    )(z_fast_real, z_fast_imag, z_slow_real, z_slow_imag, zfsq, zssq,
      cwT_syn, b_syn, csq_syn,
      cwT_sem, b_sem, csq_sem)

    mesh = plsc.VectorSubcoreMesh(core_axis_name="c", subcore_axis_name="s")
    gather = pl.kernel(
        _sc_gather, mesh=mesh,
        out_type=(
            jax.ShapeDtypeStruct((B, D), jnp.float32),
            jax.ShapeDtypeStruct((B, D), jnp.float32),
        ),
        scratch_types=[
            pltpu.VMEM((_SC_CHUNK,), jnp.int32),
            pltpu.VMEM((_SC_CHUNK, D), jnp.float32),
            pltpu.SemaphoreType.DMA,
        ],
    )
    qfr, qfi = gather(cb_syn[:, :D], cb_syn[:, D:], idx_syn.reshape(B))
    qsr, qsi = gather(cb_sem[:, :D], cb_sem[:, D:], idx_sem.reshape(B))

    zq_syn = jax.lax.complex(qfr, qfi)
    zq_sem = jax.lax.complex(qsr, qsi)
    loss = loss_acc[0, 0] * ((1.0 + COMMITMENT_COST) / (B * DIM))
    return (zq_syn, zq_sem, loss, idx_syn[:, 0], idx_sem[:, 0])


# speedup vs baseline: 1.0057x; 1.0057x over previous
import jax
import jax.numpy as jnp
from jax import lax
from jax.experimental import pallas as pl
from jax.experimental.pallas import tpu as pltpu, tpu_sc as plsc

B = 16384
D = 128
DIM = 2 * D
N_SYN = 512
N_SEM = 1024
CTX_GATE_STRENGTH = 2.0
COMMITMENT_COST = 0.25

BLOCK_B = 2048


def _vq_block(zb, zsq, cwT, b, csq, k):
    p = jax.lax.dot_general(zb, cwT, (((1,), (0,)), ((), ())),
                            preferred_element_type=jnp.float32)
    zc = p[:, :k]
    d = (zsq + csq) - 2.0 * zc
    logits = p[:, k:] + b
    m = jnp.max(logits, axis=1, keepdims=True)
    e = jnp.exp(logits - m)
    bias = CTX_GATE_STRENGTH * (e / jnp.sum(e, axis=1, keepdims=True))
    dtot = d - bias
    dmin = jnp.min(dtot, axis=1, keepdims=True)
    lane = jax.lax.broadcasted_iota(jnp.int32, dtot.shape, 1)
    idx = jnp.min(jnp.where(dtot == dmin, lane, k), axis=1)
    bias_at = jnp.sum(jnp.where(lane == idx[:, None], bias, 0.0), axis=1)
    loss_part = jnp.sum(dmin[:, 0] + bias_at)
    return idx, loss_part


def _tc_kernel(zfr_ref, zfi_ref, zsr_ref, zsi_ref, zfsq_ref, zssq_ref,
               cwT_syn_ref, b_syn_ref, csq_syn_ref,
               cwT_sem_ref, b_sem_ref, csq_sem_ref,
               idx_syn_ref, idx_sem_ref, loss_ref):
    zf = jnp.concatenate([zfr_ref[...], zfi_ref[...]], axis=1)
    zs = jnp.concatenate([zsr_ref[...], zsi_ref[...]], axis=1)
    i_syn, l_syn = _vq_block(zf.astype(jnp.bfloat16), zfsq_ref[...],
                             cwT_syn_ref[...], b_syn_ref[...],
                             csq_syn_ref[...], N_SYN)
    i_sem, l_sem = _vq_block(zs.astype(jnp.bfloat16), zssq_ref[...],
                             cwT_sem_ref[...], b_sem_ref[...],
                             csq_sem_ref[...], N_SEM)
    idx_syn_ref[...] = i_syn[:, None]
    idx_sem_ref[...] = i_sem[:, None]

    @pl.when(pl.program_id(0) == 0)
    def _init():
        loss_ref[...] = jnp.zeros_like(loss_ref)

    loss_ref[...] += l_syn + l_sem


_SC_CHUNK = 256


def _sc_gather(tab_re_hbm, tab_im_hbm, idx_hbm, out_re_hbm, out_im_hbm,
               idx_v, rows_re_v, rows_im_v, sem_re, sem_im):
    info = plsc.get_sparse_core_info()
    nw = info.num_cores * info.num_subcores
    wid = lax.axis_index("s") * info.num_cores + lax.axis_index("c")
    per_w = B // nw
    base = wid * per_w
    nchunks = per_w // _SC_CHUNK
    for c in range(nchunks):
        off = base + c * _SC_CHUNK
        sl = pl.ds(off, _SC_CHUNK)
        pltpu.sync_copy(idx_hbm.at[sl], idx_v)
        cp_re = pltpu.async_copy(tab_re_hbm.at[idx_v], rows_re_v, sem_re)
        cp_im = pltpu.async_copy(tab_im_hbm.at[idx_v], rows_im_v, sem_im)
        cp_re.wait()
        pltpu.sync_copy(rows_re_v, out_re_hbm.at[sl])
        cp_im.wait()
        pltpu.sync_copy(rows_im_v, out_im_hbm.at[sl])


def kernel(z_fast_real, z_fast_imag, z_slow_real, z_slow_imag,
           cb_syn, cb_sem, W_ctx_syn, b_ctx_syn, W_ctx_sem, b_ctx_sem):
    cwT_syn = jnp.concatenate([cb_syn.T, W_ctx_syn.T], axis=1).astype(jnp.bfloat16)
    cwT_sem = jnp.concatenate([cb_sem.T, W_ctx_sem.T], axis=1).astype(jnp.bfloat16)
    csq_syn = jnp.sum(cb_syn ** 2, axis=1)[None, :]
    csq_sem = jnp.sum(cb_sem ** 2, axis=1)[None, :]
    zfsq = jnp.sum(jnp.concatenate([z_fast_real, z_fast_imag], axis=1) ** 2,
                   axis=1, keepdims=True)
    zssq = jnp.sum(jnp.concatenate([z_slow_real, z_slow_imag], axis=1) ** 2,
                   axis=1, keepdims=True)
    b_syn = b_ctx_syn[None, :]
    b_sem = b_ctx_sem[None, :]

    nb = B // BLOCK_B
    half_spec = pl.BlockSpec((BLOCK_B, D), lambda i: (i, 0))
    sq_spec = pl.BlockSpec((BLOCK_B, 1), lambda i: (i, 0))
    full = lambda shape: pl.BlockSpec(shape, lambda i: (0,) * len(shape))

    idx_syn, idx_sem, loss_acc = pl.pallas_call(
        _tc_kernel,
        grid=(nb,),
        in_specs=[
            half_spec, half_spec, half_spec, half_spec, sq_spec, sq_spec,
            full((DIM, 2 * N_SYN)), full((1, N_SYN)), full((1, N_SYN)),
            full((DIM, 2 * N_SEM)), full((1, N_SEM)), full((1, N_SEM)),
        ],
        out_specs=(
            sq_spec,
            sq_spec,
            pl.BlockSpec((1, 1), lambda i: (0, 0)),
        ),
        out_shape=(
            jax.ShapeDtypeStruct((B, 1), jnp.int32),
            jax.ShapeDtypeStruct((B, 1), jnp.int32),
            jax.ShapeDtypeStruct((1, 1), jnp.float32),
        ),
    )(z_fast_real, z_fast_imag, z_slow_real, z_slow_imag, zfsq, zssq,
      cwT_syn, b_syn, csq_syn,
      cwT_sem, b_sem, csq_sem)

    mesh = plsc.VectorSubcoreMesh(core_axis_name="c", subcore_axis_name="s")
    gather = pl.kernel(
        _sc_gather, mesh=mesh,
        out_type=(
            jax.ShapeDtypeStruct((B, D), jnp.float32),
            jax.ShapeDtypeStruct((B, D), jnp.float32),
        ),
        scratch_types=[
            pltpu.VMEM((_SC_CHUNK,), jnp.int32),
            pltpu.VMEM((_SC_CHUNK, D), jnp.float32),
            pltpu.VMEM((_SC_CHUNK, D), jnp.float32),
            pltpu.SemaphoreType.DMA,
            pltpu.SemaphoreType.DMA,
        ],
    )
    qfr, qfi = gather(cb_syn[:, :D], cb_syn[:, D:], idx_syn.reshape(B))
    qsr, qsi = gather(cb_sem[:, :D], cb_sem[:, D:], idx_sem.reshape(B))

    zq_syn = jax.lax.complex(qfr, qfi)
    zq_sem = jax.lax.complex(qsr, qsi)
    loss = loss_acc[0, 0] * ((1.0 + COMMITMENT_COST) / (B * DIM))
    return (zq_syn, zq_sem, loss, idx_syn[:, 0], idx_sem[:, 0])


# speedup vs baseline: 1.1436x; 1.1371x over previous
import jax
import jax.numpy as jnp
from jax.experimental import pallas as pl

B = 16384
D = 128
DIM = 2 * D
N_SYN = 512
N_SEM = 1024
CTX_GATE_STRENGTH = 2.0
COMMITMENT_COST = 0.25

BLOCK_B = 2048


def _vq_block(z, zb, zsq, cwT, cb, b, csq):
    k = cb.shape[0]
    p = jax.lax.dot_general(zb, cwT, (((1,), (0,)), ((), ())),
                            preferred_element_type=jnp.float32)
    zc = p[:, :k]
    d = (zsq + csq) - 2.0 * zc
    logits = p[:, k:] + b
    m = jnp.max(logits, axis=1, keepdims=True)
    e = jnp.exp(logits - m)
    bias = CTX_GATE_STRENGTH * (e / jnp.sum(e, axis=1, keepdims=True))
    dtot = d - bias
    dmin = jnp.min(dtot, axis=1, keepdims=True)
    lane = jax.lax.broadcasted_iota(jnp.int32, dtot.shape, 1)
    idx = jnp.min(jnp.where(dtot == dmin, lane, k), axis=1)
    onehot = (lane == idx[:, None]).astype(jnp.bfloat16)
    zq = jax.lax.dot_general(onehot, cb, (((1,), (0,)), ((), ())),
                             preferred_element_type=jnp.float32)
    r = zq - z
    return zq, idx, jnp.sum(r * r)


def _fused_kernel(zfr_ref, zfi_ref, zsr_ref, zsi_ref, zfsq_ref, zssq_ref,
                  cwT_syn_ref, cb_syn_ref, b_syn_ref, csq_syn_ref,
                  cwT_sem_ref, cb_sem_ref, b_sem_ref, csq_sem_ref,
                  qfr_ref, qfi_ref, qsr_ref, qsi_ref,
                  idx_syn_ref, idx_sem_ref, loss_ref):
    zf = jnp.concatenate([zfr_ref[...], zfi_ref[...]], axis=1)
    zs = jnp.concatenate([zsr_ref[...], zsi_ref[...]], axis=1)
    zfb = zf.astype(jnp.bfloat16)
    zsb = zs.astype(jnp.bfloat16)
    qf, i_syn, l_syn = _vq_block(zf, zfb, zfsq_ref[...],
                                 cwT_syn_ref[...], cb_syn_ref[...],
                                 b_syn_ref[...], csq_syn_ref[...])
    qs, i_sem, l_sem = _vq_block(zs, zsb, zssq_ref[...],
                                 cwT_sem_ref[...], cb_sem_ref[...],
                                 b_sem_ref[...], csq_sem_ref[...])
    qfr_ref[...] = qf[:, :D]
    qfi_ref[...] = qf[:, D:]
    qsr_ref[...] = qs[:, :D]
    qsi_ref[...] = qs[:, D:]
    idx_syn_ref[...] = i_syn[:, None]
    idx_sem_ref[...] = i_sem[:, None]

    @pl.when(pl.program_id(0) == 0)
    def _init():
        loss_ref[...] = jnp.zeros_like(loss_ref)

    loss_ref[...] += l_syn + l_sem


def kernel(z_fast_real, z_fast_imag, z_slow_real, z_slow_imag,
           cb_syn, cb_sem, W_ctx_syn, b_ctx_syn, W_ctx_sem, b_ctx_sem):
    cwT_syn = jnp.concatenate([cb_syn.T, W_ctx_syn.T], axis=1).astype(jnp.bfloat16)
    cwT_sem = jnp.concatenate([cb_sem.T, W_ctx_sem.T], axis=1).astype(jnp.bfloat16)
    cb_syn_b = cb_syn.astype(jnp.bfloat16)
    cb_sem_b = cb_sem.astype(jnp.bfloat16)
    csq_syn = jnp.sum(cb_syn ** 2, axis=1)[None, :]
    csq_sem = jnp.sum(cb_sem ** 2, axis=1)[None, :]
    zfsq = jnp.sum(jnp.concatenate([z_fast_real, z_fast_imag], axis=1) ** 2,
                   axis=1, keepdims=True)
    zssq = jnp.sum(jnp.concatenate([z_slow_real, z_slow_imag], axis=1) ** 2,
                   axis=1, keepdims=True)
    b_syn = b_ctx_syn[None, :]
    b_sem = b_ctx_sem[None, :]

    nb = B // BLOCK_B
    half_spec = pl.BlockSpec((BLOCK_B, D), lambda i: (i, 0))
    row_spec = pl.BlockSpec((BLOCK_B, DIM), lambda i: (i, 0))
    sq_spec = pl.BlockSpec((BLOCK_B, 1), lambda i: (i, 0))
    full = lambda shape: pl.BlockSpec(shape, lambda i: (0,) * len(shape))

    out_shapes = (
        jax.ShapeDtypeStruct((B, D), jnp.float32),
        jax.ShapeDtypeStruct((B, D), jnp.float32),
        jax.ShapeDtypeStruct((B, D), jnp.float32),
        jax.ShapeDtypeStruct((B, D), jnp.float32),
        jax.ShapeDtypeStruct((B, 1), jnp.int32),
        jax.ShapeDtypeStruct((B, 1), jnp.int32),
        jax.ShapeDtypeStruct((1, 1), jnp.float32),
    )
    out_specs = (
        half_spec,
        half_spec,
        half_spec,
        half_spec,
        sq_spec,
        sq_spec,
        pl.BlockSpec((1, 1), lambda i: (0, 0)),
    )
    in_specs = [
        half_spec, half_spec, half_spec, half_spec, sq_spec, sq_spec,
        full((DIM, 2 * N_SYN)), full((N_SYN, DIM)),
        full((1, N_SYN)), full((1, N_SYN)),
        full((DIM, 2 * N_SEM)), full((N_SEM, DIM)),
        full((1, N_SEM)), full((1, N_SEM)),
    ]

    qfr, qfi, qsr, qsi, idx_syn, idx_sem, loss_acc = pl.pallas_call(
        _fused_kernel,
        grid=(nb,),
        in_specs=in_specs,
        out_specs=out_specs,
        out_shape=out_shapes,
    )(z_fast_real, z_fast_imag, z_slow_real, z_slow_imag, zfsq, zssq,
      cwT_syn, cb_syn_b, b_syn, csq_syn,
      cwT_sem, cb_sem_b, b_sem, csq_sem)

    zq_syn = jax.lax.complex(qfr, qfi)
    zq_sem = jax.lax.complex(qsr, qsi)
    loss = loss_acc[0, 0] * ((1.0 + COMMITMENT_COST) / (B * DIM))
    return (zq_syn, zq_sem, loss, idx_syn[:, 0], idx_sem[:, 0])
